# baseline (device time: 507314 ns/iter reference)
import jax
import jax.numpy as jnp
from jax import lax
from jax.experimental import pallas as pl
from jax.experimental.pallas import tpu as pltpu

N_DEV = 16


def _silu(y):
    yc = jnp.clip(y, -60.0, 60.0)
    return y / (1.0 + jnp.exp(-yc))


def kernel(x, w_mat):
    m, k_shard = x.shape
    _, n = w_mat.shape
    chunk = m // N_DEV

    def body(x_ref, w_ref, out_ref, comm_ref, send_sems, recv_sems, ready_sem):
        my = lax.axis_index("i")
        right = lax.rem(my + 1, N_DEV)
        left = lax.rem(my + N_DEV - 1, N_DEV)

        barrier_sem = pltpu.get_barrier_semaphore()
        for nbr in (left, right):
            pl.semaphore_signal(
                barrier_sem, inc=1,
                device_id=(nbr,), device_id_type=pl.DeviceIdType.MESH,
            )
        pl.semaphore_wait(barrier_sem, 2)

        out_ref[:, :] = jnp.dot(
            x_ref[:, :], w_ref[:, :], preferred_element_type=jnp.float32
        )

        def rows(c):
            return pl.ds(c * chunk, chunk)

        comm_ref[0, :, :] = out_ref[rows(my), :]

        H = 2 * (N_DEV - 1)
        for h in range(H):
            send_slot = h % 2
            recv_slot = (h + 1) % 2

            if h >= 1:
                pl.semaphore_wait(ready_sem, 1)

            rdma = pltpu.make_async_remote_copy(
                src_ref=comm_ref.at[send_slot],
                dst_ref=comm_ref.at[recv_slot],
                send_sem=send_sems.at[send_slot],
                recv_sem=recv_sems.at[recv_slot],
                device_id=(right,),
                device_id_type=pl.DeviceIdType.MESH,
            )
            rdma.start()
            rdma.wait()

            if h < N_DEV - 1:
                rc = lax.rem(my + 2 * N_DEV - 1 - h, N_DEV)
                if h < N_DEV - 2:
                    comm_ref[recv_slot, :, :] = (
                        comm_ref[recv_slot, :, :] + out_ref[rows(rc), :]
                    )
                else:
                    red = _silu(comm_ref[recv_slot, :, :] + out_ref[rows(rc), :])
                    out_ref[rows(rc), :] = red
                    comm_ref[recv_slot, :, :] = red
            else:
                c = lax.rem(my + 2 * N_DEV - 1 - h, N_DEV)
                out_ref[rows(c), :] = comm_ref[recv_slot, :, :]

            if h < H - 1:
                pl.semaphore_signal(
                    ready_sem, inc=1,
                    device_id=(left,), device_id_type=pl.DeviceIdType.MESH,
                )

    return pl.pallas_call(
        body,
        out_shape=jax.ShapeDtypeStruct((m, n), jnp.float32),
        in_specs=[
            pl.BlockSpec(memory_space=pltpu.VMEM),
            pl.BlockSpec(memory_space=pltpu.VMEM),
        ],
        out_specs=pl.BlockSpec(memory_space=pltpu.VMEM),
        scratch_shapes=[
            pltpu.VMEM((2, chunk, n), jnp.float32),
            pltpu.SemaphoreType.DMA((2,)),
            pltpu.SemaphoreType.DMA((2,)),
            pltpu.SemaphoreType.REGULAR,
        ],
        compiler_params=pltpu.CompilerParams(collective_id=0),
    )(x, w_mat)


# device time: 338182 ns/iter; 1.5001x vs baseline; 1.5001x over previous
import jax
import jax.numpy as jnp
from jax import lax
from jax.experimental import pallas as pl
from jax.experimental.pallas import tpu as pltpu

N_DEV = 16


def _silu(y):
    yc = jnp.clip(y, -60.0, 60.0)
    return y / (1.0 + jnp.exp(-yc))


def kernel(x, w_mat):
    m, k_shard = x.shape
    _, n = w_mat.shape
    chunk = m // N_DEV
    n2 = n // 2

    def body(x_ref, w_ref, out_ref,
             comm_r, comm_l, send_r, recv_r, send_l, recv_l,
             ready_r, ready_l):
        my = lax.axis_index("i")
        right = lax.rem(my + 1, N_DEV)
        left = lax.rem(my + N_DEV - 1, N_DEV)

        barrier_sem = pltpu.get_barrier_semaphore()
        for nbr in (left, right):
            pl.semaphore_signal(
                barrier_sem, inc=1,
                device_id=(nbr,), device_id_type=pl.DeviceIdType.MESH,
            )
        pl.semaphore_wait(barrier_sem, 2)

        out_ref[:, :] = jnp.dot(
            x_ref[:, :], w_ref[:, :], preferred_element_type=jnp.float32
        )

        def rows(c):
            return pl.ds(c * chunk, chunk)

        comm_r[0, :, :] = out_ref[rows(my), :n2]
        comm_l[0, :, :] = out_ref[rows(my), n2:]

        H = 2 * (N_DEV - 1)
        for h in range(H):
            ss = h % 2
            rs = (h + 1) % 2

            if h >= 1:
                pl.semaphore_wait(ready_r, 1)
                pl.semaphore_wait(ready_l, 1)

            rdma_r = pltpu.make_async_remote_copy(
                src_ref=comm_r.at[ss], dst_ref=comm_r.at[rs],
                send_sem=send_r.at[ss], recv_sem=recv_r.at[rs],
                device_id=(right,), device_id_type=pl.DeviceIdType.MESH,
            )
            rdma_l = pltpu.make_async_remote_copy(
                src_ref=comm_l.at[ss], dst_ref=comm_l.at[rs],
                send_sem=send_l.at[ss], recv_sem=recv_l.at[rs],
                device_id=(left,), device_id_type=pl.DeviceIdType.MESH,
            )
            rdma_r.start()
            rdma_l.start()

            rdma_r.wait()
            cr = lax.rem(my + 2 * N_DEV - 1 - h, N_DEV)
            if h < N_DEV - 2:
                comm_r[rs, :, :] = comm_r[rs, :, :] + out_ref[rows(cr), :n2]
            elif h == N_DEV - 2:
                red = _silu(comm_r[rs, :, :] + out_ref[rows(cr), :n2])
                out_ref[rows(cr), :n2] = red
                comm_r[rs, :, :] = red
            else:
                out_ref[rows(cr), :n2] = comm_r[rs, :, :]
            if h < H - 1:
                pl.semaphore_signal(
                    ready_r, inc=1,
                    device_id=(left,), device_id_type=pl.DeviceIdType.MESH,
                )

            rdma_l.wait()
            cl = lax.rem(my + 1 + h, N_DEV)
            if h < N_DEV - 2:
                comm_l[rs, :, :] = comm_l[rs, :, :] + out_ref[rows(cl), n2:]
            elif h == N_DEV - 2:
                red = _silu(comm_l[rs, :, :] + out_ref[rows(cl), n2:])
                out_ref[rows(cl), n2:] = red
                comm_l[rs, :, :] = red
            else:
                cl_ag = lax.rem(my + h + 1, N_DEV)
                out_ref[rows(cl_ag), n2:] = comm_l[rs, :, :]
            if h < H - 1:
                pl.semaphore_signal(
                    ready_l, inc=1,
                    device_id=(right,), device_id_type=pl.DeviceIdType.MESH,
                )

    return pl.pallas_call(
        body,
        out_shape=jax.ShapeDtypeStruct((m, n), jnp.float32),
        in_specs=[
            pl.BlockSpec(memory_space=pltpu.VMEM),
            pl.BlockSpec(memory_space=pltpu.VMEM),
        ],
        out_specs=pl.BlockSpec(memory_space=pltpu.VMEM),
        scratch_shapes=[
            pltpu.VMEM((2, chunk, n2), jnp.float32),
            pltpu.VMEM((2, chunk, n2), jnp.float32),
            pltpu.SemaphoreType.DMA((2,)),
            pltpu.SemaphoreType.DMA((2,)),
            pltpu.SemaphoreType.DMA((2,)),
            pltpu.SemaphoreType.DMA((2,)),
            pltpu.SemaphoreType.REGULAR,
            pltpu.SemaphoreType.REGULAR,
        ],
        compiler_params=pltpu.CompilerParams(collective_id=0),
    )(x, w_mat)


# device time: 257827 ns/iter; 1.9677x vs baseline; 1.3117x over previous
import jax
import jax.numpy as jnp
from jax import lax
from jax.experimental import pallas as pl
from jax.experimental.pallas import tpu as pltpu

N_DEV = 16

RING = (0, 4, 8, 12, 15, 11, 7, 3, 2, 6, 10, 14, 13, 9, 5, 1)
POS_OF = tuple(RING.index(d) for d in range(N_DEV))
RIGHT_OF = tuple(RING[(RING.index(d) + 1) % N_DEV] for d in range(N_DEV))
LEFT_OF = tuple(RING[(RING.index(d) - 1) % N_DEV] for d in range(N_DEV))


def _lut(table, idx):
    r = jnp.int32(table[0])
    for k in range(1, len(table)):
        r = jnp.where(idx == k, jnp.int32(table[k]), r)
    return r


def _silu(y):
    yc = jnp.clip(y, -60.0, 60.0)
    return y / (1.0 + jnp.exp(-yc))


def kernel(x, w_mat):
    m, k_shard = x.shape
    _, n = w_mat.shape
    chunk = m // N_DEV
    n2 = n // 2

    def body(x_ref, w_ref, out_ref,
             comm_r, comm_l, send_r, recv_r, send_l, recv_l,
             ready_r, ready_l):
        my = lax.axis_index("i")
        p = _lut(POS_OF, my)
        right = _lut(RIGHT_OF, my)
        left = _lut(LEFT_OF, my)

        barrier_sem = pltpu.get_barrier_semaphore()
        for nbr in (left, right):
            pl.semaphore_signal(
                barrier_sem, inc=1,
                device_id=(nbr,), device_id_type=pl.DeviceIdType.MESH,
            )
        pl.semaphore_wait(barrier_sem, 2)

        out_ref[:, :] = jnp.dot(
            x_ref[:, :], w_ref[:, :], preferred_element_type=jnp.float32
        )

        def rows(c):
            return pl.ds(c * chunk, chunk)

        comm_r[0, :, :] = out_ref[rows(p), :n2]
        comm_l[0, :, :] = out_ref[rows(p), n2:]

        H = 2 * (N_DEV - 1)
        for h in range(H):
            ss = h % 2
            rs = (h + 1) % 2

            if h >= 1:
                pl.semaphore_wait(ready_r, 1)
                pl.semaphore_wait(ready_l, 1)

            rdma_r = pltpu.make_async_remote_copy(
                src_ref=comm_r.at[ss], dst_ref=comm_r.at[rs],
                send_sem=send_r.at[ss], recv_sem=recv_r.at[rs],
                device_id=(right,), device_id_type=pl.DeviceIdType.MESH,
            )
            rdma_l = pltpu.make_async_remote_copy(
                src_ref=comm_l.at[ss], dst_ref=comm_l.at[rs],
                send_sem=send_l.at[ss], recv_sem=recv_l.at[rs],
                device_id=(left,), device_id_type=pl.DeviceIdType.MESH,
            )
            rdma_r.start()
            rdma_l.start()

            rdma_r.wait()
            cr = lax.rem(p + 2 * N_DEV - 1 - h, N_DEV)
            if h < N_DEV - 2:
                comm_r[rs, :, :] = comm_r[rs, :, :] + out_ref[rows(cr), :n2]
            elif h == N_DEV - 2:
                red = _silu(comm_r[rs, :, :] + out_ref[rows(cr), :n2])
                out_ref[rows(cr), :n2] = red
                comm_r[rs, :, :] = red
            else:
                out_ref[rows(cr), :n2] = comm_r[rs, :, :]
            if h < H - 1:
                pl.semaphore_signal(
                    ready_r, inc=1,
                    device_id=(left,), device_id_type=pl.DeviceIdType.MESH,
                )

            rdma_l.wait()
            cl = lax.rem(p + 1 + h, N_DEV)
            if h < N_DEV - 2:
                comm_l[rs, :, :] = comm_l[rs, :, :] + out_ref[rows(cl), n2:]
            elif h == N_DEV - 2:
                red = _silu(comm_l[rs, :, :] + out_ref[rows(cl), n2:])
                out_ref[rows(cl), n2:] = red
                comm_l[rs, :, :] = red
            else:
                cl_ag = lax.rem(p + h + 1, N_DEV)
                out_ref[rows(cl_ag), n2:] = comm_l[rs, :, :]
            if h < H - 1:
                pl.semaphore_signal(
                    ready_l, inc=1,
                    device_id=(right,), device_id_type=pl.DeviceIdType.MESH,
                )

    return pl.pallas_call(
        body,
        out_shape=jax.ShapeDtypeStruct((m, n), jnp.float32),
        in_specs=[
            pl.BlockSpec(memory_space=pltpu.VMEM),
            pl.BlockSpec(memory_space=pltpu.VMEM),
        ],
        out_specs=pl.BlockSpec(memory_space=pltpu.VMEM),
        scratch_shapes=[
            pltpu.VMEM((2, chunk, n2), jnp.float32),
            pltpu.VMEM((2, chunk, n2), jnp.float32),
            pltpu.SemaphoreType.DMA((2,)),
            pltpu.SemaphoreType.DMA((2,)),
            pltpu.SemaphoreType.DMA((2,)),
            pltpu.SemaphoreType.DMA((2,)),
            pltpu.SemaphoreType.REGULAR,
            pltpu.SemaphoreType.REGULAR,
        ],
        compiler_params=pltpu.CompilerParams(collective_id=0),
    )(x, w_mat)


# device time: 185079 ns/iter; 2.7411x vs baseline; 1.3931x over previous
import jax
import jax.numpy as jnp
from jax import lax
from jax.experimental import pallas as pl
from jax.experimental.pallas import tpu as pltpu

N_DEV = 16

RING = (0, 4, 8, 12, 15, 11, 7, 3, 2, 6, 10, 14, 13, 9, 5, 1)
POS_OF = tuple(RING.index(d) for d in range(N_DEV))
RIGHT_OF = tuple(RING[(RING.index(d) + 1) % N_DEV] for d in range(N_DEV))
LEFT_OF = tuple(RING[(RING.index(d) - 1) % N_DEV] for d in range(N_DEV))

N_SUB = 2
N_PIPE = 2 * N_SUB
H = 2 * (N_DEV - 1)


def _lut(table, idx):
    r = jnp.int32(table[0])
    for k in range(1, len(table)):
        r = jnp.where(idx == k, jnp.int32(table[k]), r)
    return r


def _silu(y):
    yc = jnp.clip(y, -60.0, 60.0)
    return y / (1.0 + jnp.exp(-yc))


def kernel(x, w_mat):
    m, k_shard = x.shape
    _, n = w_mat.shape
    chunk = m // N_DEV
    nq = n // N_PIPE

    def body(x_ref, w_ref, out_ref, comm, send_sems, recv_sems, ready_sems):
        my = lax.axis_index("i")
        p = _lut(POS_OF, my)
        right = _lut(RIGHT_OF, my)
        left = _lut(LEFT_OF, my)

        barrier_sem = pltpu.get_barrier_semaphore()
        for nbr in (left, right):
            pl.semaphore_signal(
                barrier_sem, inc=1,
                device_id=(nbr,), device_id_type=pl.DeviceIdType.MESH,
            )
        pl.semaphore_wait(barrier_sem, 2)

        out_ref[:, :] = jnp.dot(
            x_ref[:, :], w_ref[:, :], preferred_element_type=jnp.float32
        )

        def rows(c):
            return pl.ds(c * chunk, chunk)

        pipes = (
            dict(i=0, to=right, frm=left, lo=0, dirR=True),
            dict(i=1, to=left, frm=right, lo=2 * nq, dirR=False),
            dict(i=2, to=right, frm=left, lo=nq, dirR=True),
            dict(i=3, to=left, frm=right, lo=3 * nq, dirR=False),
        )

        def cols(pipe):
            return pl.ds(pipe["lo"], nq)

        def chunk_idx(pipe, h):
            if pipe["dirR"]:
                return lax.rem(p + 2 * N_DEV - 1 - h, N_DEV)
            return lax.rem(p + 1 + h, N_DEV)

        def hop_rdma(pipe, h):
            i, ss, rs = pipe["i"], h % 2, (h + 1) % 2
            return pltpu.make_async_remote_copy(
                src_ref=comm.at[i, ss], dst_ref=comm.at[i, rs],
                send_sem=send_sems.at[i, ss], recv_sem=recv_sems.at[i, rs],
                device_id=(pipe["to"],), device_id_type=pl.DeviceIdType.MESH,
            )

        inflight = [None] * N_PIPE
        for pipe in pipes:
            comm[pipe["i"], 0, :, :] = out_ref[rows(p), cols(pipe)]
        for pipe in pipes:
            r = hop_rdma(pipe, 0)
            r.start()
            inflight[pipe["i"]] = r

        for h in range(H):
            rs = (h + 1) % 2
            for pipe in pipes:
                i = pipe["i"]
                inflight[i].wait()
                c = chunk_idx(pipe, h)
                if h < N_DEV - 2:
                    comm[i, rs, :, :] = (
                        comm[i, rs, :, :] + out_ref[rows(c), cols(pipe)]
                    )
                elif h == N_DEV - 2:
                    red = _silu(comm[i, rs, :, :] + out_ref[rows(c), cols(pipe)])
                    out_ref[rows(c), cols(pipe)] = red
                    comm[i, rs, :, :] = red
                else:
                    out_ref[rows(c), cols(pipe)] = comm[i, rs, :, :]
                if h < H - 1:
                    pl.semaphore_signal(
                        ready_sems.at[i], inc=1,
                        device_id=(pipe["frm"],),
                        device_id_type=pl.DeviceIdType.MESH,
                    )
                    pl.semaphore_wait(ready_sems.at[i], 1)
                    nxt = hop_rdma(pipe, h + 1)
                    nxt.start()
                    inflight[i] = nxt

    return pl.pallas_call(
        body,
        out_shape=jax.ShapeDtypeStruct((m, n), jnp.float32),
        in_specs=[
            pl.BlockSpec(memory_space=pltpu.VMEM),
            pl.BlockSpec(memory_space=pltpu.VMEM),
        ],
        out_specs=pl.BlockSpec(memory_space=pltpu.VMEM),
        scratch_shapes=[
            pltpu.VMEM((N_PIPE, 2, chunk, nq), jnp.float32),
            pltpu.SemaphoreType.DMA((N_PIPE, 2)),
            pltpu.SemaphoreType.DMA((N_PIPE, 2)),
            pltpu.SemaphoreType.REGULAR((N_PIPE,)),
        ],
        compiler_params=pltpu.CompilerParams(collective_id=0),
    )(x, w_mat)


# device time: 181324 ns/iter; 2.7978x vs baseline; 1.0207x over previous
import jax
import jax.numpy as jnp
from jax import lax
from jax.experimental import pallas as pl
from jax.experimental.pallas import tpu as pltpu

N_DEV = 16

RING = (0, 4, 8, 12, 15, 11, 7, 3, 2, 6, 10, 14, 13, 9, 5, 1)
POS_OF = tuple(RING.index(d) for d in range(N_DEV))
RIGHT_OF = tuple(RING[(RING.index(d) + 1) % N_DEV] for d in range(N_DEV))
LEFT_OF = tuple(RING[(RING.index(d) - 1) % N_DEV] for d in range(N_DEV))

N_SUB = 2
N_PIPE = 2 * N_SUB
H = 2 * (N_DEV - 1)


def _lut(table, idx):
    r = jnp.int32(table[0])
    for k in range(1, len(table)):
        r = jnp.where(idx == k, jnp.int32(table[k]), r)
    return r


def _silu(y):
    yc = jnp.clip(y, -60.0, 60.0)
    return y / (1.0 + jnp.exp(-yc))


def kernel(x, w_mat):
    m, k_shard = x.shape
    _, n = w_mat.shape
    chunk = m // N_DEV
    nq = n // N_PIPE

    def body(x_ref, w_ref, out_ref, comm, send_sems, recv_sems, ready_sems):
        my = lax.axis_index("i")
        p = _lut(POS_OF, my)
        right = _lut(RIGHT_OF, my)
        left = _lut(LEFT_OF, my)

        barrier_sem = pltpu.get_barrier_semaphore()
        for nbr in (left, right):
            pl.semaphore_signal(
                barrier_sem, inc=1,
                device_id=(nbr,), device_id_type=pl.DeviceIdType.MESH,
            )
        pl.semaphore_wait(barrier_sem, 2)

        def rows(c):
            return pl.ds(c * chunk, chunk)

        def compute_chunk(c):
            out_ref[rows(c), :] = jnp.dot(
                x_ref[rows(c), :], w_ref[:, :],
                preferred_element_type=jnp.float32,
            )

        pipes = (
            dict(i=0, to=right, frm=left, lo=0, dirR=True),
            dict(i=1, to=left, frm=right, lo=2 * nq, dirR=False),
            dict(i=2, to=right, frm=left, lo=nq, dirR=True),
            dict(i=3, to=left, frm=right, lo=3 * nq, dirR=False),
        )

        def cols(pipe):
            return pl.ds(pipe["lo"], nq)

        def chunk_idx(pipe, h):
            if pipe["dirR"]:
                return lax.rem(p + 2 * N_DEV - 1 - h, N_DEV)
            return lax.rem(p + 1 + h, N_DEV)

        def hop_rdma(pipe, h):
            i, ss, rs = pipe["i"], h % 2, (h + 1) % 2
            return pltpu.make_async_remote_copy(
                src_ref=comm.at[i, ss], dst_ref=comm.at[i, rs],
                send_sem=send_sems.at[i, ss], recv_sem=recv_sems.at[i, rs],
                device_id=(pipe["to"],), device_id_type=pl.DeviceIdType.MESH,
            )

        compute_chunk(p)
        inflight = [None] * N_PIPE
        for pipe in pipes:
            comm[pipe["i"], 0, :, :] = out_ref[rows(p), cols(pipe)]
        for pipe in pipes:
            r = hop_rdma(pipe, 0)
            r.start()
            inflight[pipe["i"]] = r
        compute_chunk(lax.rem(p + 1, N_DEV))
        compute_chunk(lax.rem(p + N_DEV - 1, N_DEV))

        for h in range(H):
            rs = (h + 1) % 2
            j = h + 2
            if j <= N_DEV // 2:
                compute_chunk(lax.rem(p + j, N_DEV))
                if j < N_DEV // 2:
                    compute_chunk(lax.rem(p + N_DEV - j, N_DEV))
            for pipe in pipes:
                i = pipe["i"]
                inflight[i].wait()
                c = chunk_idx(pipe, h)
                if h < N_DEV - 2:
                    comm[i, rs, :, :] = (
                        comm[i, rs, :, :] + out_ref[rows(c), cols(pipe)]
                    )
                elif h == N_DEV - 2:
                    red = _silu(comm[i, rs, :, :] + out_ref[rows(c), cols(pipe)])
                    out_ref[rows(c), cols(pipe)] = red
                    comm[i, rs, :, :] = red
                else:
                    out_ref[rows(c), cols(pipe)] = comm[i, rs, :, :]
                if h < H - 1:
                    pl.semaphore_signal(
                        ready_sems.at[i], inc=1,
                        device_id=(pipe["frm"],),
                        device_id_type=pl.DeviceIdType.MESH,
                    )
                    pl.semaphore_wait(ready_sems.at[i], 1)
                    nxt = hop_rdma(pipe, h + 1)
                    nxt.start()
                    inflight[i] = nxt

    return pl.pallas_call(
        body,
        out_shape=jax.ShapeDtypeStruct((m, n), jnp.float32),
        in_specs=[
            pl.BlockSpec(memory_space=pltpu.VMEM),
            pl.BlockSpec(memory_space=pltpu.VMEM),
        ],
        out_specs=pl.BlockSpec(memory_space=pltpu.VMEM),
        scratch_shapes=[
            pltpu.VMEM((N_PIPE, 2, chunk, nq), jnp.float32),
            pltpu.SemaphoreType.DMA((N_PIPE, 2)),
            pltpu.SemaphoreType.DMA((N_PIPE, 2)),
            pltpu.SemaphoreType.REGULAR((N_PIPE,)),
        ],
        compiler_params=pltpu.CompilerParams(collective_id=0),
    )(x, w_mat)
